# Initial kernel scaffold; baseline (speedup 1.0000x reference)
import jax
import jax.numpy as jnp
from jax.experimental import pallas as pl


def kernel(input, input_lengths, table, W):
    raise NotImplementedError("write your pallas kernel here")


# speedup vs baseline: 2.1416x; 2.1416x over previous
import functools

import jax
import jax.numpy as jnp
from jax import lax
from jax.experimental import pallas as pl
from jax.experimental.pallas import tpu as pltpu
from jax.experimental.pallas import tpu_sc as plsc

B, L, V, D = 16384, 50, 1000000, 32

NC, NS = 2, 16
NW = NC * NS
BPW = B // NW
ROWS = (BPW * L) // 128
JG = BPW // 16

_VR = 250000
_VB = 2000


def _tc_project_body(tab_ref, wb_ref, t_ref):
    t_ref[...] = jnp.dot(tab_ref[...], wb_ref[...],
                         preferred_element_type=jnp.float32)


def _project_table(table, W):
    w = W.reshape(D)
    wb = (jnp.eye(4, dtype=jnp.float32)[:, None, :] * w[None, :, None]
          ).reshape(4 * D, 4)
    tab_v = table.reshape(_VR, 4 * D)
    t2d = pl.pallas_call(
        _tc_project_body,
        grid=(_VR // _VB,),
        in_specs=[
            pl.BlockSpec((_VB, 4 * D), lambda i: (i, 0)),
            pl.BlockSpec((4 * D, 4), lambda i: (0, 0)),
        ],
        out_specs=pl.BlockSpec((_VB, 4), lambda i: (i, 0)),
        out_shape=jax.ShapeDtypeStruct((_VR, 4), jnp.float32),
    )(tab_v, wb)
    return t2d.reshape(V)


@functools.partial(
    pl.kernel,
    out_type=jax.ShapeDtypeStruct((B,), jnp.float32),
    mesh=plsc.VectorSubcoreMesh(core_axis_name="c", subcore_axis_name="s"),
    scratch_types=[
        pltpu.VMEM((BPW * L,), jnp.int32),
        pltpu.VMEM((BPW * L,), jnp.float32),
        pltpu.VMEM((BPW,), jnp.float32),
        pltpu.SemaphoreType.DMA,
    ],
)
def _sc_gather_sum(idx_hbm, t_hbm, out_hbm, idx_v, vals_v, acc_v, sem):
    wid = lax.axis_index("s") * NC + lax.axis_index("c")
    pltpu.sync_copy(idx_hbm.at[wid], idx_v)
    pltpu.async_copy(t_hbm.at[idx_v], vals_v, sem).wait()
    for jg in range(JG):
        base = jg * 16

        def body(l, acc, base=base):
            return acc + vals_v[pl.ds(l * BPW + base, 16)]

        acc = lax.fori_loop(0, L, body, jnp.zeros((16,), jnp.float32))
        acc_v[pl.ds(base, 16)] = acc
    pltpu.sync_copy(acc_v, out_hbm.at[pl.ds(wid * BPW, BPW)])


def kernel(input, input_lengths, table, W):
    del input_lengths
    t = _project_table(table, W)
    idx = input.astype(jnp.int32).reshape(NW, BPW, L)
    idx = idx.transpose(0, 2, 1).reshape(NW, BPW * L)
    out = _sc_gather_sum(idx, t)
    return out.reshape(B, 1)
